# baseline (speedup 1.0000x reference)
import jax
import jax.numpy as jnp
from jax import lax
from jax.experimental import pallas as pl

_B, _N, _K, _D = 256, 64, 1024, 256


def _vq_body(wq_ref, cb_ref, w_ref, idx_ref, oh_ref):
    n = pl.program_id(0)
    wq = wq_ref[...]
    cb = cb_ref[0]

    w2 = jnp.sum(wq * wq, axis=1, keepdims=True)
    c2 = lax.dot_general(jnp.ones((1, _D), jnp.float32), cb * cb,
                         (((1,), (1,)), ((), ())),
                         precision=lax.Precision.HIGHEST,
                         preferred_element_type=jnp.float32)
    cross = lax.dot_general(wq, cb, (((1,), (1,)), ((), ())),
                            preferred_element_type=jnp.float32)
    dist = w2 - 2.0 * cross + c2

    m = jnp.min(dist, axis=1, keepdims=True)
    kiota = lax.broadcasted_iota(jnp.int32, (_B, _K), 1)
    idx_col = jnp.min(jnp.where(dist == m, kiota, _K), axis=1,
                      keepdims=True)
    oh = (kiota == idx_col).astype(jnp.float32)

    w = lax.dot_general(oh, cb, (((1,), (0,)), ((), ())),
                        precision=lax.Precision.HIGHEST,
                        preferred_element_type=jnp.float32)

    w_ref[...] = w
    oh_ref[...] = oh

    niota = lax.broadcasted_iota(jnp.int32, (_B, _N), 1)
    contrib = jnp.where(niota == n, idx_col, 0)

    @pl.when(n == 0)
    def _():
        idx_ref[...] = contrib

    @pl.when(n != 0)
    def _():
        idx_ref[...] += contrib


def kernel(w_q, codebook):
    wq2d = w_q.reshape(_B, _N * _D)
    w2d, idx, oh2d = pl.pallas_call(
        _vq_body,
        grid=(_N,),
        in_specs=[
            pl.BlockSpec((_B, _D), lambda n: (0, n)),
            pl.BlockSpec((1, _K, _D), lambda n: (n, 0, 0)),
        ],
        out_specs=[
            pl.BlockSpec((_B, _D), lambda n: (0, n)),
            pl.BlockSpec((_B, _N), lambda n: (0, 0)),
            pl.BlockSpec((_B, _K), lambda n: (0, n)),
        ],
        out_shape=[
            jax.ShapeDtypeStruct((_B, _N * _D), jnp.float32),
            jax.ShapeDtypeStruct((_B, _N), jnp.int32),
            jax.ShapeDtypeStruct((_B, _N * _K), jnp.float32),
        ],
    )(wq2d, codebook)
    return (w2d.reshape(_B, _N, _D), idx, oh2d.reshape(_B, _N, _K))


# speedup vs baseline: 1.8222x; 1.8222x over previous
import jax
import jax.numpy as jnp
from jax import lax
from jax.experimental import pallas as pl

_B, _N, _K, _D = 256, 64, 1024, 256


def _vq_body(wq_ref, cb_ref, w_ref, idx_ref, oh_ref):
    n = pl.program_id(0)
    wq = wq_ref[...]
    cb = cb_ref[0]

    w2 = jnp.sum(wq * wq, axis=1, keepdims=True)
    c2 = jnp.sum(cb * cb, axis=1).reshape(1, _K)
    cross = lax.dot_general(wq, cb, (((1,), (1,)), ((), ())),
                            preferred_element_type=jnp.float32)
    dist = w2 - 2.0 * cross + c2

    m = jnp.min(dist, axis=1, keepdims=True)
    kiota = lax.broadcasted_iota(jnp.int32, (_B, _K), 1)
    idx_col = jnp.min(jnp.where(dist == m, kiota, _K), axis=1,
                      keepdims=True)
    oh = (kiota == idx_col).astype(jnp.float32)

    w = lax.dot_general(oh, cb, (((1,), (0,)), ((), ())),
                        preferred_element_type=jnp.float32)

    w_ref[...] = w
    oh_ref[...] = oh

    niota = lax.broadcasted_iota(jnp.int32, (_B, _N), 1)
    contrib = jnp.where(niota == n, idx_col, 0)

    @pl.when(n == 0)
    def _():
        idx_ref[...] = contrib

    @pl.when(n != 0)
    def _():
        idx_ref[...] += contrib


def kernel(w_q, codebook):
    wq2d = w_q.reshape(_B, _N * _D)
    w2d, idx, oh2d = pl.pallas_call(
        _vq_body,
        grid=(_N,),
        in_specs=[
            pl.BlockSpec((_B, _D), lambda n: (0, n)),
            pl.BlockSpec((1, _K, _D), lambda n: (n, 0, 0)),
        ],
        out_specs=[
            pl.BlockSpec((_B, _D), lambda n: (0, n)),
            pl.BlockSpec((_B, _N), lambda n: (0, 0)),
            pl.BlockSpec((_B, _K), lambda n: (0, n)),
        ],
        out_shape=[
            jax.ShapeDtypeStruct((_B, _N * _D), jnp.float32),
            jax.ShapeDtypeStruct((_B, _N), jnp.int32),
            jax.ShapeDtypeStruct((_B, _N * _K), jnp.float32),
        ],
    )(wq2d, codebook)
    return (w2d.reshape(_B, _N, _D), idx, oh2d.reshape(_B, _N, _K))


# speedup vs baseline: 3.9769x; 2.1825x over previous
import jax
import jax.numpy as jnp
from jax import lax
from jax.experimental import pallas as pl

_B, _N, _K, _D = 256, 64, 1024, 256
_NT = 8
_STEPS = _N // _NT


def _vq_body(wq_ref, cb_ref, w_ref, idx_ref, oh_ref):
    i = pl.program_id(0)
    niota = lax.broadcasted_iota(jnp.int32, (_B, _N), 1)
    kiota = lax.broadcasted_iota(jnp.int32, (_B, _K), 1)

    acc = jnp.zeros((_B, _N), jnp.int32)
    for j in range(_NT):
        wq = wq_ref[:, j, :]
        cb = cb_ref[j]

        w2 = jnp.sum(wq * wq, axis=1, keepdims=True)
        c2 = jnp.sum(cb * cb, axis=1).reshape(1, _K)
        cross = lax.dot_general(wq, cb, (((1,), (1,)), ((), ())),
                                preferred_element_type=jnp.float32)
        dist = w2 - 2.0 * cross + c2

        m = jnp.min(dist, axis=1, keepdims=True)
        idx_col = jnp.min(jnp.where(dist == m, kiota, _K), axis=1,
                          keepdims=True)
        oh = (kiota == idx_col).astype(jnp.float32)

        w = lax.dot_general(oh, cb, (((1,), (0,)), ((), ())),
                            preferred_element_type=jnp.float32)

        w_ref[:, j, :] = w
        oh_ref[:, j, :] = oh
        acc += jnp.where(niota == i * _NT + j, idx_col, 0)

    @pl.when(i == 0)
    def _():
        idx_ref[...] = acc

    @pl.when(i != 0)
    def _():
        idx_ref[...] += acc


def kernel(w_q, codebook):
    return tuple(pl.pallas_call(
        _vq_body,
        grid=(_STEPS,),
        in_specs=[
            pl.BlockSpec((_B, _NT, _D), lambda i: (0, i, 0)),
            pl.BlockSpec((_NT, _K, _D), lambda i: (i, 0, 0)),
        ],
        out_specs=[
            pl.BlockSpec((_B, _NT, _D), lambda i: (0, i, 0)),
            pl.BlockSpec((_B, _N), lambda i: (0, 0)),
            pl.BlockSpec((_B, _NT, _K), lambda i: (0, i, 0)),
        ],
        out_shape=[
            jax.ShapeDtypeStruct((_B, _N, _D), jnp.float32),
            jax.ShapeDtypeStruct((_B, _N), jnp.int32),
            jax.ShapeDtypeStruct((_B, _N, _K), jnp.float32),
        ],
    )(w_q, codebook))
